# Initial kernel scaffold; baseline (speedup 1.0000x reference)
import jax
import jax.numpy as jnp
from jax.experimental import pallas as pl


def kernel(x, W_in, b_in, W1, g1, be1, W2, g2, be2, W4, g4, be4, W5, g5, be5, W6, g6, be6, W7, g7, be7):
    raise NotImplementedError("write your pallas kernel here")


# speedup vs baseline: 4.0901x; 4.0901x over previous
import functools
from typing import Any

import jax
import jax.numpy as jnp
from jax.experimental import pallas as pl

K = 16
GROUPS = 4
EPS = 1e-5
INTERPRET = False


def _proj_body(x_ref, w_ref, b_ref, o_ref):
    x = x_ref[0]
    o_ref[0] = jnp.dot(x, w_ref[...], preferred_element_type=jnp.float32) + b_ref[...]


def proj(x, W_in, b_in):
    B, N, _ = x.shape
    O = W_in.shape[0]
    return pl.pallas_call(
        _proj_body,
        grid=(B,),
        in_specs=[
            pl.BlockSpec((1, N, 3), lambda b: (b, 0, 0)),
            pl.BlockSpec((3, O), lambda b: (0, 0)),
            pl.BlockSpec((1, O), lambda b: (0, 0)),
        ],
        out_specs=pl.BlockSpec((1, N, O), lambda b: (b, 0, 0)),
        out_shape=jax.ShapeDtypeStruct((B, N, O), jnp.float32),
        interpret=INTERPRET,
    )(x, W_in.T, b_in.reshape(1, O))


def _edge_body(cq_ref, ckt_ref, xq_ref, xk_ref, wt_ref,
               ymax_ref, ymin_ref, st_ref, *, O):
    i = pl.program_id(1)
    cq = cq_ref[0]
    ckt = ckt_ref[0]
    xq = xq_ref[0]
    xk = xk_ref[0]
    qsq = jnp.sum(cq * cq, axis=1, keepdims=True)
    ksq = jnp.sum(ckt * ckt, axis=0, keepdims=True)
    D = qsq - 2.0 * jnp.dot(cq, ckt, preferred_element_type=jnp.float32) + ksq
    lane = jax.lax.broadcasted_iota(jnp.int32, D.shape, 1)
    BIGI = jnp.int32(2 ** 30)
    INF = jnp.float32(jnp.inf)
    acc_max = jnp.full((cq.shape[0], O), -INF, jnp.float32)
    acc_min = jnp.full((cq.shape[0], O), INF, jnp.float32)
    s1 = jnp.zeros((1, O), jnp.float32)
    s2 = jnp.zeros((1, O), jnp.float32)
    wt = wt_ref[...]

    def step(_, carry):
        D, acc_max, acc_min, s1, s2 = carry
        m = jnp.min(D, axis=1, keepdims=True)
        j = jnp.min(jnp.where(D <= m, lane, BIGI), axis=1, keepdims=True)
        oh = lane == j
        F = jax.lax.dot_general(oh.astype(jnp.float32), xk,
                                (((1,), (0,)), ((), ())),
                                precision=jax.lax.Precision.HIGHEST,
                                preferred_element_type=jnp.float32)
        e = jnp.concatenate([F - xq, xq], axis=1)
        y = jnp.dot(e, wt, preferred_element_type=jnp.float32)
        acc_max = jnp.maximum(acc_max, y)
        acc_min = jnp.minimum(acc_min, y)
        s1 = s1 + jnp.sum(y, axis=0, keepdims=True)
        s2 = s2 + jnp.sum(y * y, axis=0, keepdims=True)
        D = jnp.where(oh, INF, D)
        return D, acc_max, acc_min, s1, s2

    D, acc_max, acc_min, s1, s2 = jax.lax.fori_loop(
        0, K, step, (D, acc_max, acc_min, s1, s2))
    ymax_ref[0] = acc_max
    ymin_ref[0] = acc_min

    @pl.when(i == 0)
    def _():
        st_ref[...] = jnp.zeros_like(st_ref)

    st_ref[0, 0:1, :] += s1
    st_ref[0, 1:2, :] += s2


def edge_stage(cq, ckt, xq, xk, W, Tq=256):
    B, Nq, C = xq.shape
    Nk = xk.shape[1]
    O = W.shape[0]
    Tq = min(Tq, Nq)
    NB = Nq // Tq
    WT = W.T
    grid = (B, NB)
    ymax, ymin, st = pl.pallas_call(
        functools.partial(_edge_body, O=O),
        grid=grid,
        in_specs=[
            pl.BlockSpec((1, Tq, 3), lambda b, i: (b, i, 0)),
            pl.BlockSpec((1, 3, Nk), lambda b, i: (b, 0, 0)),
            pl.BlockSpec((1, Tq, C), lambda b, i: (b, i, 0)),
            pl.BlockSpec((1, Nk, C), lambda b, i: (b, 0, 0)),
            pl.BlockSpec((2 * C, O), lambda b, i: (0, 0)),
        ],
        out_specs=[
            pl.BlockSpec((1, Tq, O), lambda b, i: (b, i, 0)),
            pl.BlockSpec((1, Tq, O), lambda b, i: (b, i, 0)),
            pl.BlockSpec((1, 8, O), lambda b, i: (b, 0, 0)),
        ],
        out_shape=[
            jax.ShapeDtypeStruct((B, Nq, O), jnp.float32),
            jax.ShapeDtypeStruct((B, Nq, O), jnp.float32),
            jax.ShapeDtypeStruct((B, 8, O), jnp.float32),
        ],
        interpret=INTERPRET,
    )(cq, ckt, xq, xk, WT)
    return ymax, ymin, st


def _fin_body(ymax_ref, ymin_ref, st_ref, g_ref, be_ref, o_ref, *, O, cnt):
    s1 = st_ref[0, 0:1, :]
    s2 = st_ref[0, 1:2, :]
    gs = O // GROUPS
    co = jax.lax.broadcasted_iota(jnp.int32, (O, O), 0) // gs
    co2 = jax.lax.broadcasted_iota(jnp.int32, (O, O), 1) // gs
    Gm = (co == co2).astype(jnp.float32)
    mu = jnp.dot(s1, Gm, preferred_element_type=jnp.float32) / cnt
    e2 = jnp.dot(s2, Gm, preferred_element_type=jnp.float32) / cnt
    var = e2 - mu * mu
    g = g_ref[...]
    be = be_ref[...]
    a = g * jax.lax.rsqrt(var + EPS)
    b = be - mu * a
    v = jnp.where(a >= 0, ymax_ref[0], ymin_ref[0])
    z = v * a + b
    o_ref[0] = jnp.where(z >= 0, z, 0.2 * z)


def finalize(ymax, ymin, st, gamma, beta):
    B, Nq, O = ymax.shape
    cnt = float(Nq * K * (O // GROUPS))
    return pl.pallas_call(
        functools.partial(_fin_body, O=O, cnt=cnt),
        grid=(B,),
        in_specs=[
            pl.BlockSpec((1, Nq, O), lambda b: (b, 0, 0)),
            pl.BlockSpec((1, Nq, O), lambda b: (b, 0, 0)),
            pl.BlockSpec((1, 8, O), lambda b: (b, 0, 0)),
            pl.BlockSpec((1, O), lambda b: (0, 0)),
            pl.BlockSpec((1, O), lambda b: (0, 0)),
        ],
        out_specs=pl.BlockSpec((1, Nq, O), lambda b: (b, 0, 0)),
        out_shape=jax.ShapeDtypeStruct((B, Nq, O), jnp.float32),
        interpret=INTERPRET,
    )(ymax, ymin, st, gamma.reshape(1, O), beta.reshape(1, O))


def _fps_body(x3_ref, idx_ref, *, B, N, npoint):
    X = x3_ref[...]
    lane = jax.lax.broadcasted_iota(jnp.int32, (1, N), 1)
    lane_np = jax.lax.broadcasted_iota(jnp.int32, (1, npoint), 1)
    BIGI = jnp.int32(2 ** 30)

    def body(i, carry):
        dists, last, idxs = carry
        oh = (lane == last)[:, None, :]
        cur = jnp.sum(jnp.where(oh, X, 0.0), axis=2, keepdims=True)
        d = jnp.sum((X - cur) ** 2, axis=1)
        dists = jnp.minimum(dists, d)
        mx = jnp.max(dists, axis=1, keepdims=True)
        nxt = jnp.min(jnp.where(dists >= mx, lane, BIGI), axis=1, keepdims=True)
        idxs = idxs + jnp.where(lane_np == i, nxt, 0)
        return dists, nxt, idxs

    dists0 = jnp.full((B, N), 1e10, jnp.float32)
    last0 = jnp.zeros((B, 1), jnp.int32)
    idxs0 = jnp.zeros((B, npoint), jnp.int32)
    _, _, idxs = jax.lax.fori_loop(1, npoint, body, (dists0, last0, idxs0))
    idx_ref[...] = idxs


def fps(coor, npoint):
    B, N, _ = coor.shape
    x3 = jnp.transpose(coor, (0, 2, 1))
    return pl.pallas_call(
        functools.partial(_fps_body, B=B, N=N, npoint=npoint),
        in_specs=[pl.BlockSpec((B, 3, N), lambda: (0, 0, 0))],
        out_specs=pl.BlockSpec((B, npoint), lambda: (0, 0)),
        out_shape=jax.ShapeDtypeStruct((B, npoint), jnp.int32),
        interpret=INTERPRET,
    )(x3)


def _dsg_body(idx_ref, comb_ref, o_ref, *, N):
    idxrow = idx_ref[0]
    comb = comb_ref[0]
    sub = jax.lax.broadcasted_iota(jnp.int32, (N, idxrow.shape[1]), 0)
    oh = (sub == idxrow).astype(jnp.float32)
    o_ref[0] = jax.lax.dot_general(
        oh, comb, (((0,), (0,)), ((), ())),
        precision=jax.lax.Precision.HIGHEST,
        preferred_element_type=jnp.float32)


def dsgather(comb, idx, Tp=256):
    B, N, F = comb.shape
    npoint = idx.shape[1]
    Tp = min(Tp, npoint)
    NB = npoint // Tp
    idx3 = idx.reshape(B, 1, npoint)
    return pl.pallas_call(
        functools.partial(_dsg_body, N=N),
        grid=(B, NB),
        in_specs=[
            pl.BlockSpec((1, 1, Tp), lambda b, i: (b, 0, i)),
            pl.BlockSpec((1, N, F), lambda b, i: (b, 0, 0)),
        ],
        out_specs=pl.BlockSpec((1, Tp, F), lambda b, i: (b, i, 0)),
        out_shape=jax.ShapeDtypeStruct((B, npoint, F), jnp.float32),
        interpret=INTERPRET,
    )(idx3, comb)


def _stage(cq, ck, xq, xk, W, g, be, Tq=256):
    ckt = jnp.transpose(ck, (0, 2, 1))
    ymax, ymin, st = edge_stage(cq, ckt, xq, xk, W, Tq=Tq)
    return finalize(ymax, ymin, st, g, be)


def kernel(x, W_in, b_in, W1, g1, be1, W2, g2, be2, W4, g4, be4,
           W5, g5, be5, W6, g6, be6, W7, g7, be7):
    coor0 = x
    f0 = proj(x, W_in, b_in)
    f1 = _stage(coor0, coor0, f0, f0, W1, g1, be1)

    idx1 = fps(coor0, 1024)
    nc = dsgather(jnp.concatenate([coor0, f1], axis=2), idx1)
    coor1, fq1 = nc[:, :, :3], nc[:, :, 3:]
    s1f = _stage(coor1, coor0, fq1, f1, W2, g2, be2)

    idx2 = fps(coor1, 512)
    nc = dsgather(jnp.concatenate([coor1, s1f], axis=2), idx2)
    coor2, fq2 = nc[:, :, :3], nc[:, :, 3:]
    f4 = _stage(coor2, coor1, fq2, s1f, W4, g4, be4)
    s2f = _stage(coor2, coor2, f4, f4, W5, g5, be5)

    idx3 = fps(coor2, 256)
    nc = dsgather(jnp.concatenate([coor2, s2f], axis=2), idx3)
    coor3, fq3 = nc[:, :, :3], nc[:, :, 3:]
    f6 = _stage(coor3, coor2, fq3, s2f, W6, g6, be6)
    s3f = _stage(coor3, coor3, f6, f6, W7, g7, be7)

    return coor3, s3f


# speedup vs baseline: 5.8374x; 1.4272x over previous
import functools
from typing import Any

import jax
import jax.numpy as jnp
from jax.experimental import pallas as pl

K = 16
GROUPS = 4
EPS = 1e-5
INTERPRET = False


def _proj_body(x_ref, w_ref, b_ref, o_ref):
    x = x_ref[0]
    o_ref[0] = jnp.dot(x, w_ref[...], preferred_element_type=jnp.float32) + b_ref[...]


def proj(x, W_in, b_in):
    B, N, _ = x.shape
    O = W_in.shape[0]
    return pl.pallas_call(
        _proj_body,
        grid=(B,),
        in_specs=[
            pl.BlockSpec((1, N, 3), lambda b: (b, 0, 0)),
            pl.BlockSpec((3, O), lambda b: (0, 0)),
            pl.BlockSpec((1, O), lambda b: (0, 0)),
        ],
        out_specs=pl.BlockSpec((1, N, O), lambda b: (b, 0, 0)),
        out_shape=jax.ShapeDtypeStruct((B, N, O), jnp.float32),
        interpret=INTERPRET,
    )(x, W_in.T, b_in.reshape(1, O))


def _edge_body(cq_ref, ckt_ref, xq_ref, xkh_ref, xkm_ref, xkl_ref, wt_ref,
               ymax_ref, ymin_ref, st_ref, *, O):
    i = pl.program_id(1)
    cq = cq_ref[0]
    ckt = ckt_ref[0]
    xq = xq_ref[0]
    qsq = jnp.sum(cq * cq, axis=1, keepdims=True)
    ksq = jnp.sum(ckt * ckt, axis=0, keepdims=True)
    D = qsq - 2.0 * jnp.dot(cq, ckt, preferred_element_type=jnp.float32) + ksq
    lane = jax.lax.broadcasted_iota(jnp.int32, D.shape, 1)
    BIGI = jnp.int32(2 ** 30)
    INF = jnp.float32(jnp.inf)
    acc_max = jnp.full((cq.shape[0], O), -INF, jnp.float32)
    acc_min = jnp.full((cq.shape[0], O), INF, jnp.float32)
    s1 = jnp.zeros((1, O), jnp.float32)
    s2 = jnp.zeros((1, O), jnp.float32)
    wt = wt_ref[...]

    def step(_, carry):
        D, acc_max, acc_min, s1, s2 = carry
        m = jnp.min(D, axis=1, keepdims=True)
        j = jnp.min(jnp.where(D <= m, lane, BIGI), axis=1, keepdims=True)
        oh = lane == j
        ohb = oh.astype(jnp.bfloat16)
        dims = (((1,), (0,)), ((), ()))
        Fh = jax.lax.dot_general(ohb, xkh_ref[0], dims, preferred_element_type=jnp.float32)
        Fm = jax.lax.dot_general(ohb, xkm_ref[0], dims, preferred_element_type=jnp.float32)
        Fl = jax.lax.dot_general(ohb, xkl_ref[0], dims, preferred_element_type=jnp.float32)
        F = (Fh + Fm) + Fl
        e = jnp.concatenate([F - xq, xq], axis=1)
        y = jnp.dot(e, wt, preferred_element_type=jnp.float32)
        acc_max = jnp.maximum(acc_max, y)
        acc_min = jnp.minimum(acc_min, y)
        s1 = s1 + jnp.sum(y, axis=0, keepdims=True)
        s2 = s2 + jnp.sum(y * y, axis=0, keepdims=True)
        D = jnp.where(oh, INF, D)
        return D, acc_max, acc_min, s1, s2

    D, acc_max, acc_min, s1, s2 = jax.lax.fori_loop(
        0, K, step, (D, acc_max, acc_min, s1, s2))
    ymax_ref[0] = acc_max
    ymin_ref[0] = acc_min

    @pl.when(i == 0)
    def _():
        st_ref[...] = jnp.zeros_like(st_ref)

    st_ref[0, 0:1, :] += s1
    st_ref[0, 1:2, :] += s2


def edge_stage(cq, ckt, xq, xk, W, Tq=256):
    B, Nq, C = xq.shape
    Nk = xk.shape[1]
    O = W.shape[0]
    Tq = min(Tq, Nq)
    NB = Nq // Tq
    WT = W.T
    xkh = xk.astype(jnp.bfloat16)
    r = xk - xkh.astype(jnp.float32)
    xkm = r.astype(jnp.bfloat16)
    xkl = (r - xkm.astype(jnp.float32)).astype(jnp.bfloat16)
    grid = (B, NB)
    ymax, ymin, st = pl.pallas_call(
        functools.partial(_edge_body, O=O),
        grid=grid,
        in_specs=[
            pl.BlockSpec((1, Tq, 3), lambda b, i: (b, i, 0)),
            pl.BlockSpec((1, 3, Nk), lambda b, i: (b, 0, 0)),
            pl.BlockSpec((1, Tq, C), lambda b, i: (b, i, 0)),
            pl.BlockSpec((1, Nk, C), lambda b, i: (b, 0, 0)),
            pl.BlockSpec((1, Nk, C), lambda b, i: (b, 0, 0)),
            pl.BlockSpec((1, Nk, C), lambda b, i: (b, 0, 0)),
            pl.BlockSpec((2 * C, O), lambda b, i: (0, 0)),
        ],
        out_specs=[
            pl.BlockSpec((1, Tq, O), lambda b, i: (b, i, 0)),
            pl.BlockSpec((1, Tq, O), lambda b, i: (b, i, 0)),
            pl.BlockSpec((1, 8, O), lambda b, i: (b, 0, 0)),
        ],
        out_shape=[
            jax.ShapeDtypeStruct((B, Nq, O), jnp.float32),
            jax.ShapeDtypeStruct((B, Nq, O), jnp.float32),
            jax.ShapeDtypeStruct((B, 8, O), jnp.float32),
        ],
        interpret=INTERPRET,
    )(cq, ckt, xq, xkh, xkm, xkl, WT)
    return ymax, ymin, st


def _fin_body(ymax_ref, ymin_ref, st_ref, g_ref, be_ref, o_ref, *, O, cnt):
    s1 = st_ref[0, 0:1, :]
    s2 = st_ref[0, 1:2, :]
    gs = O // GROUPS
    co = jax.lax.broadcasted_iota(jnp.int32, (O, O), 0) // gs
    co2 = jax.lax.broadcasted_iota(jnp.int32, (O, O), 1) // gs
    Gm = (co == co2).astype(jnp.float32)
    mu = jnp.dot(s1, Gm, preferred_element_type=jnp.float32) / cnt
    e2 = jnp.dot(s2, Gm, preferred_element_type=jnp.float32) / cnt
    var = e2 - mu * mu
    g = g_ref[...]
    be = be_ref[...]
    a = g * jax.lax.rsqrt(var + EPS)
    b = be - mu * a
    v = jnp.where(a >= 0, ymax_ref[0], ymin_ref[0])
    z = v * a + b
    o_ref[0] = jnp.where(z >= 0, z, 0.2 * z)


def finalize(ymax, ymin, st, gamma, beta):
    B, Nq, O = ymax.shape
    cnt = float(Nq * K * (O // GROUPS))
    return pl.pallas_call(
        functools.partial(_fin_body, O=O, cnt=cnt),
        grid=(B,),
        in_specs=[
            pl.BlockSpec((1, Nq, O), lambda b: (b, 0, 0)),
            pl.BlockSpec((1, Nq, O), lambda b: (b, 0, 0)),
            pl.BlockSpec((1, 8, O), lambda b: (b, 0, 0)),
            pl.BlockSpec((1, O), lambda b: (0, 0)),
            pl.BlockSpec((1, O), lambda b: (0, 0)),
        ],
        out_specs=pl.BlockSpec((1, Nq, O), lambda b: (b, 0, 0)),
        out_shape=jax.ShapeDtypeStruct((B, Nq, O), jnp.float32),
        interpret=INTERPRET,
    )(ymax, ymin, st, gamma.reshape(1, O), beta.reshape(1, O))


def _fps_body(x3_ref, idx_ref, *, B, N, npoint):
    X = x3_ref[...]
    lane = jax.lax.broadcasted_iota(jnp.int32, (1, N), 1)
    lane_np = jax.lax.broadcasted_iota(jnp.int32, (1, npoint), 1)
    BIGI = jnp.int32(2 ** 30)

    def body(i, carry):
        dists, last, idxs = carry
        oh = (lane == last)[:, None, :]
        cur = jnp.sum(jnp.where(oh, X, 0.0), axis=2, keepdims=True)
        d = jnp.sum((X - cur) ** 2, axis=1)
        dists = jnp.minimum(dists, d)
        mx = jnp.max(dists, axis=1, keepdims=True)
        nxt = jnp.min(jnp.where(dists >= mx, lane, BIGI), axis=1, keepdims=True)
        idxs = idxs + jnp.where(lane_np == i, nxt, 0)
        return dists, nxt, idxs

    dists0 = jnp.full((B, N), 1e10, jnp.float32)
    last0 = jnp.zeros((B, 1), jnp.int32)
    idxs0 = jnp.zeros((B, npoint), jnp.int32)
    _, _, idxs = jax.lax.fori_loop(1, npoint, body, (dists0, last0, idxs0))
    idx_ref[...] = idxs


def fps(coor, npoint):
    B, N, _ = coor.shape
    x3 = jnp.transpose(coor, (0, 2, 1))
    return pl.pallas_call(
        functools.partial(_fps_body, B=B, N=N, npoint=npoint),
        in_specs=[pl.BlockSpec((B, 3, N), lambda: (0, 0, 0))],
        out_specs=pl.BlockSpec((B, npoint), lambda: (0, 0)),
        out_shape=jax.ShapeDtypeStruct((B, npoint), jnp.int32),
        interpret=INTERPRET,
    )(x3)


def _dsg_body(idx_ref, comb_ref, o_ref, *, N):
    idxrow = idx_ref[0]
    comb = comb_ref[0]
    sub = jax.lax.broadcasted_iota(jnp.int32, (N, idxrow.shape[1]), 0)
    oh = (sub == idxrow).astype(jnp.float32)
    o_ref[0] = jax.lax.dot_general(
        oh, comb, (((0,), (0,)), ((), ())),
        precision=jax.lax.Precision.HIGHEST,
        preferred_element_type=jnp.float32)


def dsgather(comb, idx, Tp=256):
    B, N, F = comb.shape
    npoint = idx.shape[1]
    Tp = min(Tp, npoint)
    NB = npoint // Tp
    idx3 = idx.reshape(B, 1, npoint)
    return pl.pallas_call(
        functools.partial(_dsg_body, N=N),
        grid=(B, NB),
        in_specs=[
            pl.BlockSpec((1, 1, Tp), lambda b, i: (b, 0, i)),
            pl.BlockSpec((1, N, F), lambda b, i: (b, 0, 0)),
        ],
        out_specs=pl.BlockSpec((1, Tp, F), lambda b, i: (b, i, 0)),
        out_shape=jax.ShapeDtypeStruct((B, npoint, F), jnp.float32),
        interpret=INTERPRET,
    )(idx3, comb)


def _stage(cq, ck, xq, xk, W, g, be, Tq=256):
    ckt = jnp.transpose(ck, (0, 2, 1))
    ymax, ymin, st = edge_stage(cq, ckt, xq, xk, W, Tq=Tq)
    return finalize(ymax, ymin, st, g, be)


def kernel(x, W_in, b_in, W1, g1, be1, W2, g2, be2, W4, g4, be4,
           W5, g5, be5, W6, g6, be6, W7, g7, be7):
    coor0 = x
    f0 = proj(x, W_in, b_in)
    f1 = _stage(coor0, coor0, f0, f0, W1, g1, be1)

    idx1 = fps(coor0, 1024)
    nc = dsgather(jnp.concatenate([coor0, f1], axis=2), idx1)
    coor1, fq1 = nc[:, :, :3], nc[:, :, 3:]
    s1f = _stage(coor1, coor0, fq1, f1, W2, g2, be2)

    idx2 = fps(coor1, 512)
    nc = dsgather(jnp.concatenate([coor1, s1f], axis=2), idx2)
    coor2, fq2 = nc[:, :, :3], nc[:, :, 3:]
    f4 = _stage(coor2, coor1, fq2, s1f, W4, g4, be4)
    s2f = _stage(coor2, coor2, f4, f4, W5, g5, be5)

    idx3 = fps(coor2, 256)
    nc = dsgather(jnp.concatenate([coor2, s2f], axis=2), idx3)
    coor3, fq3 = nc[:, :, :3], nc[:, :, 3:]
    f6 = _stage(coor3, coor2, fq3, s2f, W6, g6, be6)
    s3f = _stage(coor3, coor3, f6, f6, W7, g7, be7)

    return coor3, s3f
